# baseline (speedup 1.0000x reference)
import functools

import jax
import jax.numpy as jnp
from jax import lax
from jax.experimental import pallas as pl
from jax.experimental.pallas import tpu as pltpu
from jax.experimental.pallas import tpu_sc as plsc

N = 10000
NP = 10240
RBLK = 1024
NBLK = NP // RBLK
D = 128
NC = 2
NS = 16
NW = NC * NS
CHUNK = 128
N0_CHUNKS = 80
N1_CHUNKS = 80
WIN = 16
ACC_N = 10240
PAD_DST = N


def _mesh():
    return plsc.VectorSubcoreMesh(
        core_axis_name="c", subcore_axis_name="s", num_cores=NC, num_subcores=NS
    )


@functools.lru_cache(maxsize=None)
def _deg_kernel(nmax):
    @functools.partial(
        pl.kernel,
        out_type=jax.ShapeDtypeStruct((NC, ACC_N), jnp.float32),
        mesh=_mesh(),
        scratch_types=[
            pltpu.VMEM((nmax, CHUNK), jnp.int32),
            pltpu.VMEM((CHUNK,), jnp.float32),
            pltpu.VMEM((ACC_N // NS,), jnp.float32),
            pltpu.VMEM_SHARED((ACC_N,), jnp.float32),
        ],
    )
    def deg_k(dsts_hbm, ones_hbm, zeros_hbm, out_hbm, dst_v, ones_v, z_v, acc_sh):
        cid = lax.axis_index("c")
        sid = lax.axis_index("s")
        wid = cid * NS + sid
        nch = jnp.where(cid == 0, N0_CHUNKS, N1_CHUNKS)
        seg = ACC_N // NS
        pltpu.sync_copy(dsts_hbm.at[wid], dst_v)
        pltpu.sync_copy(ones_hbm, ones_v)
        pltpu.sync_copy(zeros_hbm, z_v)
        pltpu.sync_copy(z_v, acc_sh.at[pl.ds(sid * seg, seg)])
        plsc.subcore_barrier()

        def body(j, c):
            pltpu.sync_copy(ones_v, acc_sh.at[dst_v.at[j]], add=True)
            return c

        lax.fori_loop(0, nch, body, 0)
        plsc.subcore_barrier()
        pltpu.sync_copy(
            acc_sh.at[pl.ds(sid * seg, seg)], out_hbm.at[cid, pl.ds(sid * seg, seg)]
        )

    return deg_k


@functools.lru_cache(maxsize=None)
def _agg_kernel(nmax):
    @functools.partial(
        pl.kernel,
        out_type=jax.ShapeDtypeStruct((NC, ACC_N, D), jnp.float32),
        mesh=_mesh(),
        scratch_types=[
            pltpu.VMEM((WIN, CHUNK), jnp.int32),
            pltpu.VMEM((WIN, CHUNK), jnp.int32),
            pltpu.VMEM((2, CHUNK, D), jnp.float32),
            pltpu.VMEM_SHARED((ACC_N, D), jnp.float32),
            pltpu.SemaphoreType.DMA,
            pltpu.SemaphoreType.DMA,
        ],
    )
    def agg_k(table_hbm, srcs_hbm, dsts_hbm, zacc_hbm, out_hbm,
              src_v, dst_v, rows_v, acc_sh, sem, ssem):
        cid = lax.axis_index("c")
        sid = lax.axis_index("s")
        wid = cid * NS + sid
        seg = ACC_N // NS
        with jax.named_scope("agg_setup"):
            pltpu.sync_copy(zacc_hbm, acc_sh.at[pl.ds(sid * seg, seg)])
            plsc.subcore_barrier()

        def scatter_drain():
            pltpu.make_async_copy(
                zacc_hbm.at[pl.ds(0, CHUNK)], rows_v.at[0], ssem
            ).wait()

        def win_body(w, cw):
            pltpu.sync_copy(srcs_hbm.at[wid, pl.ds(w * WIN, WIN)], src_v)
            pltpu.sync_copy(dsts_hbm.at[wid, pl.ds(w * WIN, WIN)], dst_v)

            def body(j, c):
                @pl.when(j > 0)
                def _():
                    jm = j - 1
                    pltpu.make_async_copy(
                        table_hbm.at[src_v.at[jm]], rows_v.at[jm % 2], sem
                    ).wait()
                    pltpu.async_copy(
                        rows_v.at[jm % 2], acc_sh.at[dst_v.at[jm]], ssem, add=True
                    )

                @pl.when(j > 1)
                def _():
                    scatter_drain()

                @pl.when(j < WIN)
                def _():
                    pltpu.async_copy(
                        table_hbm.at[src_v.at[j]], rows_v.at[j % 2], sem
                    )

                return c

            lax.fori_loop(0, WIN + 1, body, 0)
            scatter_drain()
            return cw

        with jax.named_scope("agg_loop"):
            lax.fori_loop(0, N0_CHUNKS // WIN, win_body, 0)
            plsc.subcore_barrier()
        with jax.named_scope("agg_writeback"):
            pltpu.sync_copy(
                acc_sh.at[pl.ds(sid * seg, seg)],
                out_hbm.at[cid, pl.ds(sid * seg, seg)],
            )

    return agg_k


def _dcol(drows):
    eye = (
        lax.broadcasted_iota(jnp.int32, (128, 128), 0)
        == lax.broadcasted_iota(jnp.int32, (128, 128), 1)
    ).astype(jnp.float32)
    cols = [
        jnp.sum(eye * drows[r : r + 1, :], axis=1, keepdims=True) for r in range(8)
    ]
    return jnp.concatenate(cols, axis=0)


def _tc1(xp, w1, degp):
    def body(x_ref, w_ref, deg_ref, h_ref, dinv_ref):
        drows = lax.rsqrt(deg_ref[0] + deg_ref[1] + 1.0)
        dinv_ref[...] = drows
        h = jnp.dot(x_ref[...], w_ref[...], preferred_element_type=jnp.float32)
        h_ref[...] = h * _dcol(drows)

    return pl.pallas_call(
        body,
        grid=(NBLK,),
        in_specs=[
            pl.BlockSpec((RBLK, D), lambda i: (i, 0)),
            pl.BlockSpec((D, D), lambda i: (0, 0)),
            pl.BlockSpec((NC, 8, 128), lambda i: (0, i, 0)),
        ],
        out_specs=(
            pl.BlockSpec((RBLK, D), lambda i: (i, 0)),
            pl.BlockSpec((8, 128), lambda i: (i, 0)),
        ),
        out_shape=(
            jax.ShapeDtypeStruct((NP, D), jnp.float32),
            jax.ShapeDtypeStruct((NP // 128, 128), jnp.float32),
        ),
    )(xp, w1, degp)


def _tc2(a, h1, dinvp, b1, w2):
    def body(a_ref, h1_ref, dinv_ref, b_ref, w_ref, h2_ref):
        dc = _dcol(dinv_ref[...])
        z = dc * (a_ref[0] + a_ref[1] + h1_ref[...]) + b_ref[...]
        z = jnp.maximum(z, 0.0)
        h2_ref[...] = (
            jnp.dot(z, w_ref[...], preferred_element_type=jnp.float32) * dc
        )

    return pl.pallas_call(
        body,
        grid=(NBLK,),
        in_specs=[
            pl.BlockSpec((NC, RBLK, D), lambda i: (0, i, 0)),
            pl.BlockSpec((RBLK, D), lambda i: (i, 0)),
            pl.BlockSpec((8, 128), lambda i: (i, 0)),
            pl.BlockSpec((1, D), lambda i: (0, 0)),
            pl.BlockSpec((D, D), lambda i: (0, 0)),
        ],
        out_specs=pl.BlockSpec((RBLK, D), lambda i: (i, 0)),
        out_shape=jax.ShapeDtypeStruct((NP, D), jnp.float32),
    )(a, h1, dinvp, b1, w2)


def _tc3(a, h2, dinvp, b2):
    def body(a_ref, h2_ref, dinv_ref, b_ref, o_ref):
        dc = _dcol(dinv_ref[...])
        o_ref[...] = dc * (a_ref[0] + a_ref[1] + h2_ref[...]) + b_ref[...]

    return pl.pallas_call(
        body,
        grid=(NBLK,),
        in_specs=[
            pl.BlockSpec((NC, RBLK, D), lambda i: (0, i, 0)),
            pl.BlockSpec((RBLK, D), lambda i: (i, 0)),
            pl.BlockSpec((8, 128), lambda i: (i, 0)),
            pl.BlockSpec((1, D), lambda i: (0, 0)),
        ],
        out_specs=pl.BlockSpec((RBLK, D), lambda i: (i, 0)),
        out_shape=jax.ShapeDtypeStruct((N, D), jnp.float32),
    )(a, h2, dinvp, b2)


def kernel(x, edge_index, W1, b1, W2, b2):
    E = edge_index.shape[1]
    tot_chunks = NS * (N0_CHUNKS + N1_CHUNKS)
    assert tot_chunks * CHUNK >= E
    pad = tot_chunks * CHUNK - E

    pad_src = jnp.arange(pad, dtype=jnp.int32) % N
    pad_dst = N + (jnp.arange(pad, dtype=jnp.int32) % (ACC_N - N))
    ep = jnp.concatenate(
        [edge_index, jnp.stack([pad_src, pad_dst])], axis=1
    ).reshape(2, NW, N0_CHUNKS, CHUNK)
    src = ep[0]
    dst = ep[1]
    ones = jnp.ones((CHUNK,), jnp.float32)
    zseg = jnp.zeros((ACC_N // NS,), jnp.float32)
    zacc = jnp.zeros((ACC_N // NS, D), jnp.float32)
    xp = jnp.pad(x, ((0, NP - N), (0, 0)))

    deg_parts = _deg_kernel(N0_CHUNKS)(dst, ones, zseg)
    degp = deg_parts.reshape(NC, ACC_N // 128, 128)
    h1, dinvp = _tc1(xp, W1, degp)

    agg = _agg_kernel(N0_CHUNKS)
    a = agg(h1, src, dst, zacc)
    h2 = _tc2(a, h1, dinvp, jnp.reshape(b1, (1, D)), W2)
    a2 = agg(h2, src, dst, zacc)
    return _tc3(a2, h2, dinvp, jnp.reshape(b2, (1, D)))


# speedup vs baseline: 1.1425x; 1.1425x over previous
import functools

import jax
import jax.numpy as jnp
from jax import lax
from jax.experimental import pallas as pl
from jax.experimental.pallas import tpu as pltpu
from jax.experimental.pallas import tpu_sc as plsc

N = 10000
NP = 10240
RBLK = 1024
NBLK = NP // RBLK
D = 128
NC = 2
NS = 16
NW = NC * NS
CHUNK = 128
N0_CHUNKS = 80
N1_CHUNKS = 80
WIN = 16
ACC_N = 10240
PAD_DST = N


def _mesh():
    return plsc.VectorSubcoreMesh(
        core_axis_name="c", subcore_axis_name="s", num_cores=NC, num_subcores=NS
    )


@functools.lru_cache(maxsize=None)
def _deg_kernel(nmax):
    @functools.partial(
        pl.kernel,
        out_type=jax.ShapeDtypeStruct((NC, ACC_N), jnp.float32),
        mesh=_mesh(),
        scratch_types=[
            pltpu.VMEM((nmax, CHUNK), jnp.int32),
            pltpu.VMEM((CHUNK,), jnp.float32),
            pltpu.VMEM((ACC_N // NS,), jnp.float32),
            pltpu.VMEM_SHARED((ACC_N,), jnp.float32),
        ],
    )
    def deg_k(dsts_hbm, ones_hbm, zeros_hbm, out_hbm, dst_v, ones_v, z_v, acc_sh):
        cid = lax.axis_index("c")
        sid = lax.axis_index("s")
        wid = cid * NS + sid
        nch = jnp.where(cid == 0, N0_CHUNKS, N1_CHUNKS)
        seg = ACC_N // NS
        pltpu.sync_copy(dsts_hbm.at[wid], dst_v)
        pltpu.sync_copy(ones_hbm, ones_v)
        pltpu.sync_copy(zeros_hbm, z_v)
        pltpu.sync_copy(z_v, acc_sh.at[pl.ds(sid * seg, seg)])
        plsc.subcore_barrier()

        def body(j, c):
            pltpu.sync_copy(ones_v, acc_sh.at[dst_v.at[j]], add=True)
            return c

        lax.fori_loop(0, nch, body, 0)
        plsc.subcore_barrier()
        pltpu.sync_copy(
            acc_sh.at[pl.ds(sid * seg, seg)], out_hbm.at[cid, pl.ds(sid * seg, seg)]
        )

    return deg_k


@functools.lru_cache(maxsize=None)
def _agg_kernel(nmax):
    @functools.partial(
        pl.kernel,
        out_type=jax.ShapeDtypeStruct((NC, ACC_N, D), jnp.float32),
        mesh=_mesh(),
        scratch_types=[
            pltpu.VMEM((WIN, CHUNK), jnp.int32),
            pltpu.VMEM((WIN, CHUNK), jnp.int32),
            pltpu.VMEM((2, CHUNK, D), jnp.float32),
            pltpu.VMEM_SHARED((ACC_N, D), jnp.float32),
            pltpu.SemaphoreType.DMA,
        ],
    )
    def agg_k(table_hbm, srcs_hbm, dsts_hbm, zacc_hbm, out_hbm,
              src_v, dst_v, rows_v, acc_sh, sem):
        cid = lax.axis_index("c")
        sid = lax.axis_index("s")
        wid = cid * NS + sid
        seg = ACC_N // NS
        with jax.named_scope("agg_setup"):
            pltpu.sync_copy(zacc_hbm, acc_sh.at[pl.ds(sid * seg, seg)])
            plsc.subcore_barrier()

        def win_body(w, cw):
            pltpu.sync_copy(srcs_hbm.at[wid, pl.ds(w * WIN, WIN)], src_v)
            pltpu.sync_copy(dsts_hbm.at[wid, pl.ds(w * WIN, WIN)], dst_v)

            def body(j, c):
                @pl.when(j < WIN)
                def _():
                    pltpu.async_copy(
                        table_hbm.at[src_v.at[j]], rows_v.at[j % 2], sem
                    )

                @pl.when(j > 0)
                def _():
                    jm = j - 1
                    pltpu.make_async_copy(
                        table_hbm.at[src_v.at[jm]], rows_v.at[jm % 2], sem
                    ).wait()
                    pltpu.sync_copy(
                        rows_v.at[jm % 2], acc_sh.at[dst_v.at[jm]], add=True
                    )

                return c

            lax.fori_loop(0, WIN + 1, body, 0)
            return cw

        with jax.named_scope("agg_loop"):
            lax.fori_loop(0, N0_CHUNKS // WIN, win_body, 0)
            plsc.subcore_barrier()
        with jax.named_scope("agg_writeback"):
            pltpu.sync_copy(
                acc_sh.at[pl.ds(sid * seg, seg)],
                out_hbm.at[cid, pl.ds(sid * seg, seg)],
            )

    return agg_k


def _dcol(drows):
    eye = (
        lax.broadcasted_iota(jnp.int32, (128, 128), 0)
        == lax.broadcasted_iota(jnp.int32, (128, 128), 1)
    ).astype(jnp.float32)
    cols = [
        jnp.sum(eye * drows[r : r + 1, :], axis=1, keepdims=True) for r in range(8)
    ]
    return jnp.concatenate(cols, axis=0)


def _tc1(xp, w1, degp):
    def body(x_ref, w_ref, deg_ref, h_ref, dinv_ref):
        drows = lax.rsqrt(deg_ref[0] + deg_ref[1] + 1.0)
        dinv_ref[...] = drows
        h = jnp.dot(x_ref[...], w_ref[...], preferred_element_type=jnp.float32)
        h_ref[...] = h * _dcol(drows)

    return pl.pallas_call(
        body,
        grid=(NBLK,),
        in_specs=[
            pl.BlockSpec((RBLK, D), lambda i: (i, 0)),
            pl.BlockSpec((D, D), lambda i: (0, 0)),
            pl.BlockSpec((NC, 8, 128), lambda i: (0, i, 0)),
        ],
        out_specs=(
            pl.BlockSpec((RBLK, D), lambda i: (i, 0)),
            pl.BlockSpec((8, 128), lambda i: (i, 0)),
        ),
        out_shape=(
            jax.ShapeDtypeStruct((NP, D), jnp.float32),
            jax.ShapeDtypeStruct((NP // 128, 128), jnp.float32),
        ),
    )(xp, w1, degp)


def _tc2(a, h1, dinvp, b1, w2):
    def body(a_ref, h1_ref, dinv_ref, b_ref, w_ref, h2_ref):
        dc = _dcol(dinv_ref[...])
        z = dc * (a_ref[0] + a_ref[1] + h1_ref[...]) + b_ref[...]
        z = jnp.maximum(z, 0.0)
        h2_ref[...] = (
            jnp.dot(z, w_ref[...], preferred_element_type=jnp.float32) * dc
        )

    return pl.pallas_call(
        body,
        grid=(NBLK,),
        in_specs=[
            pl.BlockSpec((NC, RBLK, D), lambda i: (0, i, 0)),
            pl.BlockSpec((RBLK, D), lambda i: (i, 0)),
            pl.BlockSpec((8, 128), lambda i: (i, 0)),
            pl.BlockSpec((1, D), lambda i: (0, 0)),
            pl.BlockSpec((D, D), lambda i: (0, 0)),
        ],
        out_specs=pl.BlockSpec((RBLK, D), lambda i: (i, 0)),
        out_shape=jax.ShapeDtypeStruct((NP, D), jnp.float32),
    )(a, h1, dinvp, b1, w2)


def _tc3(a, h2, dinvp, b2):
    def body(a_ref, h2_ref, dinv_ref, b_ref, o_ref):
        dc = _dcol(dinv_ref[...])
        o_ref[...] = dc * (a_ref[0] + a_ref[1] + h2_ref[...]) + b_ref[...]

    return pl.pallas_call(
        body,
        grid=(NBLK,),
        in_specs=[
            pl.BlockSpec((NC, RBLK, D), lambda i: (0, i, 0)),
            pl.BlockSpec((RBLK, D), lambda i: (i, 0)),
            pl.BlockSpec((8, 128), lambda i: (i, 0)),
            pl.BlockSpec((1, D), lambda i: (0, 0)),
        ],
        out_specs=pl.BlockSpec((RBLK, D), lambda i: (i, 0)),
        out_shape=jax.ShapeDtypeStruct((N, D), jnp.float32),
    )(a, h2, dinvp, b2)


def kernel(x, edge_index, W1, b1, W2, b2):
    E = edge_index.shape[1]
    tot_chunks = NS * (N0_CHUNKS + N1_CHUNKS)
    assert tot_chunks * CHUNK >= E
    pad = tot_chunks * CHUNK - E

    pad_src = jnp.arange(pad, dtype=jnp.int32) % N
    pad_dst = N + (jnp.arange(pad, dtype=jnp.int32) % (ACC_N - N))
    ep = jnp.concatenate(
        [edge_index, jnp.stack([pad_src, pad_dst])], axis=1
    ).reshape(2, NW, N0_CHUNKS, CHUNK)
    src = ep[0]
    dst = ep[1]
    ones = jnp.ones((CHUNK,), jnp.float32)
    zseg = jnp.zeros((ACC_N // NS,), jnp.float32)
    zacc = jnp.zeros((ACC_N // NS, D), jnp.float32)
    xp = jnp.pad(x, ((0, NP - N), (0, 0)))

    deg_parts = _deg_kernel(N0_CHUNKS)(dst, ones, zseg)
    degp = deg_parts.reshape(NC, ACC_N // 128, 128)
    h1, dinvp = _tc1(xp, W1, degp)

    agg = _agg_kernel(N0_CHUNKS)
    a = agg(h1, src, dst, zacc)
    h2 = _tc2(a, h1, dinvp, jnp.reshape(b1, (1, D)), W2)
    a2 = agg(h2, src, dst, zacc)
    return _tc3(a2, h2, dinvp, jnp.reshape(b2, (1, D)))


# speedup vs baseline: 1.1877x; 1.0396x over previous
import functools

import jax
import jax.numpy as jnp
from jax import lax
from jax.experimental import pallas as pl
from jax.experimental.pallas import tpu as pltpu
from jax.experimental.pallas import tpu_sc as plsc

N = 10000
NP = 10240
RBLK = 1024
NBLK = NP // RBLK
D = 128
NC = 2
NS = 16
NW = NC * NS
CHUNK = 128
N0_CHUNKS = 80
N1_CHUNKS = 80
WIN = 40
ACC_N = 10240
PAD_DST = N


def _mesh():
    return plsc.VectorSubcoreMesh(
        core_axis_name="c", subcore_axis_name="s", num_cores=NC, num_subcores=NS
    )


@functools.lru_cache(maxsize=None)
def _deg_kernel(nmax):
    @functools.partial(
        pl.kernel,
        out_type=jax.ShapeDtypeStruct((NC, ACC_N), jnp.float32),
        mesh=_mesh(),
        scratch_types=[
            pltpu.VMEM((nmax, CHUNK), jnp.int32),
            pltpu.VMEM((CHUNK,), jnp.float32),
            pltpu.VMEM((ACC_N // NS,), jnp.float32),
            pltpu.VMEM_SHARED((ACC_N,), jnp.float32),
        ],
    )
    def deg_k(dsts_hbm, ones_hbm, zeros_hbm, out_hbm, dst_v, ones_v, z_v, acc_sh):
        cid = lax.axis_index("c")
        sid = lax.axis_index("s")
        wid = cid * NS + sid
        nch = jnp.where(cid == 0, N0_CHUNKS, N1_CHUNKS)
        seg = ACC_N // NS
        pltpu.sync_copy(dsts_hbm.at[wid], dst_v)
        pltpu.sync_copy(ones_hbm, ones_v)
        pltpu.sync_copy(zeros_hbm, z_v)
        pltpu.sync_copy(z_v, acc_sh.at[pl.ds(sid * seg, seg)])
        plsc.subcore_barrier()

        def body(j, c):
            pltpu.sync_copy(ones_v, acc_sh.at[dst_v.at[j]], add=True)
            return c

        lax.fori_loop(0, nch, body, 0)
        plsc.subcore_barrier()
        pltpu.sync_copy(
            acc_sh.at[pl.ds(sid * seg, seg)], out_hbm.at[cid, pl.ds(sid * seg, seg)]
        )

    return deg_k


@functools.lru_cache(maxsize=None)
def _agg_kernel(nmax):
    @functools.partial(
        pl.kernel,
        out_type=jax.ShapeDtypeStruct((NC, ACC_N, D), jnp.float32),
        mesh=_mesh(),
        scratch_types=[
            pltpu.VMEM((WIN, CHUNK), jnp.int32),
            pltpu.VMEM((WIN, CHUNK), jnp.int32),
            pltpu.VMEM((2, CHUNK, D), jnp.float32),
            pltpu.VMEM_SHARED((ACC_N, D), jnp.float32),
            pltpu.SemaphoreType.DMA,
        ],
    )
    def agg_k(table_hbm, srcs_hbm, dsts_hbm, zacc_hbm, out_hbm,
              src_v, dst_v, rows_v, acc_sh, sem):
        cid = lax.axis_index("c")
        sid = lax.axis_index("s")
        wid = cid * NS + sid
        seg = ACC_N // NS
        with jax.named_scope("agg_setup"):
            pltpu.sync_copy(zacc_hbm, acc_sh.at[pl.ds(sid * seg, seg)])
            plsc.subcore_barrier()

        def win_body(w, cw):
            pltpu.sync_copy(srcs_hbm.at[wid, pl.ds(w * WIN, WIN)], src_v)
            pltpu.sync_copy(dsts_hbm.at[wid, pl.ds(w * WIN, WIN)], dst_v)

            def body(j, c):
                @pl.when(j < WIN)
                def _():
                    pltpu.async_copy(
                        table_hbm.at[src_v.at[j]], rows_v.at[j % 2], sem
                    )

                @pl.when(j > 0)
                def _():
                    jm = j - 1
                    pltpu.make_async_copy(
                        table_hbm.at[src_v.at[jm]], rows_v.at[jm % 2], sem
                    ).wait()
                    pltpu.sync_copy(
                        rows_v.at[jm % 2], acc_sh.at[dst_v.at[jm]], add=True
                    )

                return c

            lax.fori_loop(0, WIN + 1, body, 0)
            return cw

        with jax.named_scope("agg_loop"):
            lax.fori_loop(0, N0_CHUNKS // WIN, win_body, 0)
            plsc.subcore_barrier()
        with jax.named_scope("agg_writeback"):
            pltpu.sync_copy(
                acc_sh.at[pl.ds(sid * seg, seg)],
                out_hbm.at[cid, pl.ds(sid * seg, seg)],
            )

    return agg_k


def _dcol(drows):
    eye = (
        lax.broadcasted_iota(jnp.int32, (128, 128), 0)
        == lax.broadcasted_iota(jnp.int32, (128, 128), 1)
    ).astype(jnp.float32)
    cols = [
        jnp.sum(eye * drows[r : r + 1, :], axis=1, keepdims=True) for r in range(8)
    ]
    return jnp.concatenate(cols, axis=0)


def _tc1(xp, w1, degp):
    def body(x_ref, w_ref, deg_ref, h_ref, dinv_ref):
        drows = lax.rsqrt(deg_ref[0] + deg_ref[1] + 1.0)
        dinv_ref[...] = drows
        h = jnp.dot(x_ref[...], w_ref[...], preferred_element_type=jnp.float32)
        h_ref[...] = h * _dcol(drows)

    return pl.pallas_call(
        body,
        grid=(NBLK,),
        in_specs=[
            pl.BlockSpec((RBLK, D), lambda i: (i, 0)),
            pl.BlockSpec((D, D), lambda i: (0, 0)),
            pl.BlockSpec((NC, 8, 128), lambda i: (0, i, 0)),
        ],
        out_specs=(
            pl.BlockSpec((RBLK, D), lambda i: (i, 0)),
            pl.BlockSpec((8, 128), lambda i: (i, 0)),
        ),
        out_shape=(
            jax.ShapeDtypeStruct((NP, D), jnp.float32),
            jax.ShapeDtypeStruct((NP // 128, 128), jnp.float32),
        ),
    )(xp, w1, degp)


def _tc2(a, h1, dinvp, b1, w2):
    def body(a_ref, h1_ref, dinv_ref, b_ref, w_ref, h2_ref):
        dc = _dcol(dinv_ref[...])
        z = dc * (a_ref[0] + a_ref[1] + h1_ref[...]) + b_ref[...]
        z = jnp.maximum(z, 0.0)
        h2_ref[...] = (
            jnp.dot(z, w_ref[...], preferred_element_type=jnp.float32) * dc
        )

    return pl.pallas_call(
        body,
        grid=(NBLK,),
        in_specs=[
            pl.BlockSpec((NC, RBLK, D), lambda i: (0, i, 0)),
            pl.BlockSpec((RBLK, D), lambda i: (i, 0)),
            pl.BlockSpec((8, 128), lambda i: (i, 0)),
            pl.BlockSpec((1, D), lambda i: (0, 0)),
            pl.BlockSpec((D, D), lambda i: (0, 0)),
        ],
        out_specs=pl.BlockSpec((RBLK, D), lambda i: (i, 0)),
        out_shape=jax.ShapeDtypeStruct((NP, D), jnp.float32),
    )(a, h1, dinvp, b1, w2)


def _tc3(a, h2, dinvp, b2):
    def body(a_ref, h2_ref, dinv_ref, b_ref, o_ref):
        dc = _dcol(dinv_ref[...])
        o_ref[...] = dc * (a_ref[0] + a_ref[1] + h2_ref[...]) + b_ref[...]

    return pl.pallas_call(
        body,
        grid=(NBLK,),
        in_specs=[
            pl.BlockSpec((NC, RBLK, D), lambda i: (0, i, 0)),
            pl.BlockSpec((RBLK, D), lambda i: (i, 0)),
            pl.BlockSpec((8, 128), lambda i: (i, 0)),
            pl.BlockSpec((1, D), lambda i: (0, 0)),
        ],
        out_specs=pl.BlockSpec((RBLK, D), lambda i: (i, 0)),
        out_shape=jax.ShapeDtypeStruct((N, D), jnp.float32),
    )(a, h2, dinvp, b2)


def kernel(x, edge_index, W1, b1, W2, b2):
    E = edge_index.shape[1]
    tot_chunks = NS * (N0_CHUNKS + N1_CHUNKS)
    assert tot_chunks * CHUNK >= E
    pad = tot_chunks * CHUNK - E

    pad_src = jnp.arange(pad, dtype=jnp.int32) % N
    pad_dst = N + (jnp.arange(pad, dtype=jnp.int32) % (ACC_N - N))
    ep = jnp.concatenate(
        [edge_index, jnp.stack([pad_src, pad_dst])], axis=1
    ).reshape(2, NW, N0_CHUNKS, CHUNK)
    src = ep[0]
    dst = ep[1]
    ones = jnp.ones((CHUNK,), jnp.float32)
    zseg = jnp.zeros((ACC_N // NS,), jnp.float32)
    zacc = jnp.zeros((ACC_N // NS, D), jnp.float32)
    xp = jnp.pad(x, ((0, NP - N), (0, 0)))

    deg_parts = _deg_kernel(N0_CHUNKS)(dst, ones, zseg)
    degp = deg_parts.reshape(NC, ACC_N // 128, 128)
    h1, dinvp = _tc1(xp, W1, degp)

    agg = _agg_kernel(N0_CHUNKS)
    a = agg(h1, src, dst, zacc)
    h2 = _tc2(a, h1, dinvp, jnp.reshape(b1, (1, D)), W2)
    a2 = agg(h2, src, dst, zacc)
    return _tc3(a2, h2, dinvp, jnp.reshape(b2, (1, D)))
